# baseline (speedup 1.0000x reference)
import functools

import jax
import jax.numpy as jnp
from jax import lax
from jax.experimental import pallas as pl
from jax.experimental.pallas import tpu as pltpu
from jax.experimental.pallas import tpu_sc as plsc

_B = 16384
_NUM_CORES = 2
_NUM_SUBCORES = 16
_NW = _NUM_CORES * _NUM_SUBCORES
_ROWS_PER_W = _B // _NW
_CHUNK = 128
_NCHUNK = _ROWS_PER_W // _CHUNK
_EDIMS = (16, 16, 8, 8, 8)
_TOTAL_IN = 64
_HIDDEN = 128
_OUT_DIM = 64


def _sc_gather_body(t0, t1, t2, t3, t4, i0, i1, i2, i3, i4,
                    o0, o1, o2, o3, o4,
                    idx_v, r0, r1, r2, r3, r4, sem):
    wid = lax.axis_index("s") * _NUM_CORES + lax.axis_index("c")
    row0 = wid * _NCHUNK
    base = wid * _ROWS_PER_W

    idx_refs = (i0, i1, i2, i3, i4)
    tables = (t0, t1, t2, t3, t4)
    rows = (r0, r1, r2, r3, r4)
    outs = (o0, o1, o2, o3, o4)

    for f in range(5):
        pltpu.sync_copy(idx_refs[f].at[pl.ds(row0, _NCHUNK)], idx_v.at[f])

    cps = []
    for f in range(5):
        for j in range(_NCHUNK):
            cps.append(pltpu.async_copy(
                tables[f].at[idx_v.at[f, j]],
                rows[f].at[pl.ds(j * _CHUNK, _CHUNK)],
                sem))
    for cp in cps:
        cp.wait()

    for f in range(5):
        pltpu.sync_copy(rows[f], outs[f].at[pl.ds(base, _ROWS_PER_W)])


_sc_gather = functools.partial(
    pl.kernel,
    mesh=plsc.VectorSubcoreMesh(core_axis_name="c", subcore_axis_name="s"),
    compiler_params=pltpu.CompilerParams(use_tc_tiling_on_sc=False),
    out_type=[jax.ShapeDtypeStruct((_B, d), jnp.float32) for d in _EDIMS],
    scratch_types=[
        pltpu.VMEM((5, _NCHUNK, _CHUNK), jnp.int32),
        pltpu.VMEM((_ROWS_PER_W, 16), jnp.float32),
        pltpu.VMEM((_ROWS_PER_W, 16), jnp.float32),
        pltpu.VMEM((_ROWS_PER_W, 8), jnp.float32),
        pltpu.VMEM((_ROWS_PER_W, 8), jnp.float32),
        pltpu.VMEM((_ROWS_PER_W, 8), jnp.float32),
        pltpu.SemaphoreType.DMA,
    ],
)(_sc_gather_body)


def _mlp_body(e0, e1, e2, e3, e4, num, w1, b1, w2, b2, out):
    x = jnp.concatenate(
        [e0[...], e1[...], e2[...], e3[...], e4[...], num[...]], axis=1)
    h = jnp.maximum(
        jnp.dot(x, w1[...], preferred_element_type=jnp.float32) + b1[...], 0.0)
    o = jnp.maximum(
        jnp.dot(h, w2[...], preferred_element_type=jnp.float32) + b2[...], 0.0)
    out[...] = o


def _mlp(e0, e1, e2, e3, e4, num, w1, b1, w2, b2):
    blk = 2048
    grid = (_B // blk,)
    row_spec = lambda d: pl.BlockSpec((blk, d), lambda i: (i, 0))
    full_spec = lambda s: pl.BlockSpec(s, lambda i: (0, 0))
    return pl.pallas_call(
        _mlp_body,
        grid=grid,
        in_specs=[
            row_spec(16), row_spec(16), row_spec(8), row_spec(8), row_spec(8),
            row_spec(8),
            full_spec((_TOTAL_IN, _HIDDEN)), full_spec((1, _HIDDEN)),
            full_spec((_HIDDEN, _OUT_DIM)), full_spec((1, _OUT_DIM)),
        ],
        out_specs=pl.BlockSpec((blk, _OUT_DIM), lambda i: (i, 0)),
        out_shape=jax.ShapeDtypeStruct((_B, _OUT_DIM), jnp.float32),
    )(e0, e1, e2, e3, e4, num, w1, b1, w2, b2)


def kernel(user_id, city, device, age_bucket, occupation, user_num,
           E_user_id, E_city, E_device, E_age_bucket, E_occupation,
           W1, b1, W2, b2):
    idx2d = [a.reshape(_NW * _NCHUNK, _CHUNK)
             for a in (user_id, city, device, age_bucket, occupation)]
    e0, e1, e2, e3, e4 = _sc_gather(
        E_user_id, E_city, E_device, E_age_bucket, E_occupation, *idx2d)
    return _mlp(e0, e1, e2, e3, e4, user_num,
                W1, b1.reshape(1, _HIDDEN), W2, b2.reshape(1, _OUT_DIM))


# speedup vs baseline: 2.9931x; 2.9931x over previous
import functools

import jax
import jax.numpy as jnp
from jax import lax
from jax.experimental import pallas as pl
from jax.experimental.pallas import tpu as pltpu
from jax.experimental.pallas import tpu_sc as plsc

_B = 16384
_NUM_CORES = 2
_NUM_SUBCORES = 16
_NW = _NUM_CORES * _NUM_SUBCORES
_RPW = _B // _NW
_CHUNK = 128
_HIDDEN = 128
_OUT_DIM = 64

_G_TABLES = ((100000, 16, 0), (1000, 8, 16), (1000, 8, 24), (1000, 8, 32))
_G_ROWS = 40


def _sc_body(ut, cf, df, af, of, iu, ic, idv, ia, io, out,
             idx_v, gidx, ubuf, rows_u, rt, sem_u, sem_g):
    wid = lax.axis_index("s") * _NUM_CORES + lax.axis_index("c")
    base = wid * _RPW
    i16 = jnp.arange(16, dtype=jnp.int32)

    for k, ref in enumerate((ic, idv, ia, io, iu)):
        pltpu.sync_copy(ref.at[pl.ds(base, _RPW)],
                        idx_v.at[pl.ds(k * _RPW, _RPW)])

    def build(r, _):
        vs = [idx_v[pl.ds(k * _RPW + r * 16, 16)] for k in range(4)]
        row = 0
        for k, (vocab, d, _o) in enumerate(_G_TABLES):
            for e in range(d):
                gidx[pl.ds((row + e) * _RPW + r * 16, 16)] = vs[k] + e * vocab
            row += d
        return 0

    lax.fori_loop(0, _RPW // 16, build, 0)

    flats = (cf, df, af, of)
    cps_g = []
    row = 0
    for k, (vocab, d, _o) in enumerate(_G_TABLES):
        for j in range(d * _RPW // _CHUNK):
            off = row * _RPW + j * _CHUNK
            cps_g.append(pltpu.async_copy(
                flats[k].at[gidx.at[pl.ds(off, _CHUNK)]],
                rt.at[pl.ds(off, _CHUNK)], sem_g))
        row += d

    def user_chunk(c, _):
        v16 = idx_v[pl.ds(4 * _RPW + c * 16, 16)]
        cps = []
        for l in range(16):
            v0 = pl.multiple_of((v16[l] >> 7) << 7, 128)
            cps.append(pltpu.async_copy(
                ut.at[pl.ds(0, 8), pl.ds(v0, 128)], ubuf.at[2 * l], sem_u))
            cps.append(pltpu.async_copy(
                ut.at[pl.ds(8, 8), pl.ds(v0, 128)], ubuf.at[2 * l + 1], sem_u))
        for cp in cps:
            cp.wait()
        for l in range(16):
            col = jnp.full((16,), 0, jnp.int32) + (v16[l] & 127)
            tsel = 2 * l + (i16 >= 8).astype(jnp.int32)
            val = plsc.load_gather(ubuf, [tsel, i16 & 7, col])
            plsc.store_scatter(rows_u, [i16 * _RPW + (c * 16 + l)], val)
        return 0

    lax.fori_loop(0, _RPW // 16, user_chunk, 0)

    for cp in cps_g:
        cp.wait()

    for e in range(16):
        pltpu.sync_copy(rows_u.at[pl.ds(e * _RPW, _RPW)],
                        out.at[e, pl.ds(base, _RPW)])
    for e in range(_G_ROWS):
        pltpu.sync_copy(rt.at[pl.ds(e * _RPW, _RPW)],
                        out.at[16 + e, pl.ds(base, _RPW)])


_sc_gather = functools.partial(
    pl.kernel,
    mesh=plsc.VectorSubcoreMesh(core_axis_name="c", subcore_axis_name="s"),
    compiler_params=pltpu.CompilerParams(use_tc_tiling_on_sc=True,
                                         needs_layout_passes=False),
    out_type=jax.ShapeDtypeStruct((56, _B), jnp.float32),
    scratch_types=[
        pltpu.VMEM((5 * _RPW,), jnp.int32),
        pltpu.VMEM((_G_ROWS * _RPW,), jnp.int32),
        pltpu.VMEM((32, 8, 128), jnp.float32),
        pltpu.VMEM((16 * _RPW,), jnp.float32),
        pltpu.VMEM((_G_ROWS * _RPW,), jnp.float32),
        pltpu.SemaphoreType.DMA,
        pltpu.SemaphoreType.DMA,
    ],
)(_sc_body)


def _mlp_body(xt, nt, w1e, w1n, b1, w2, b2, out):
    acc = lax.dot_general(w1e[...], xt[...], (((0,), (0,)), ((), ())),
                          preferred_element_type=jnp.float32)
    acc += lax.dot_general(w1n[...], nt[...], (((0,), (0,)), ((), ())),
                           preferred_element_type=jnp.float32)
    ht = jax.nn.relu(acc + b1[...])
    out[...] = jax.nn.relu(
        lax.dot_general(w2[...], ht, (((0,), (0,)), ((), ())),
                        preferred_element_type=jnp.float32) + b2[...])


def _mlp(xt, nt, w1e, w1n, b1, w2, b2):
    blk = 4096
    full = lambda s: pl.BlockSpec(s, lambda i: (0, 0))
    return pl.pallas_call(
        _mlp_body,
        grid=(_B // blk,),
        in_specs=[
            pl.BlockSpec((56, blk), lambda i: (0, i)),
            pl.BlockSpec((8, blk), lambda i: (0, i)),
            full((56, _HIDDEN)), full((8, _HIDDEN)),
            full((_HIDDEN, 1)), full((_HIDDEN, _OUT_DIM)), full((_OUT_DIM, 1)),
        ],
        out_specs=pl.BlockSpec((_OUT_DIM, blk), lambda i: (0, i)),
        out_shape=jax.ShapeDtypeStruct((_OUT_DIM, _B), jnp.float32),
    )(xt, nt, w1e, w1n, b1, w2, b2)


def kernel(user_id, city, device, age_bucket, occupation, user_num,
           E_user_id, E_city, E_device, E_age_bucket, E_occupation,
           W1, b1, W2, b2):
    xt = _sc_gather(E_user_id.T, E_city.T.reshape(-1),
                    E_device.T.reshape(-1), E_age_bucket.T.reshape(-1),
                    E_occupation.T.reshape(-1),
                    user_id, city, device, age_bucket, occupation)
    ot = _mlp(xt, user_num.T, W1[0:56], W1[56:64],
              b1.reshape(_HIDDEN, 1), W2, b2.reshape(_OUT_DIM, 1))
    return ot.T


# speedup vs baseline: 3.4877x; 1.1653x over previous
import functools

import jax
import jax.numpy as jnp
from jax import lax
from jax.experimental import pallas as pl
from jax.experimental.pallas import tpu as pltpu
from jax.experimental.pallas import tpu_sc as plsc

_B = 16384
_NUM_CORES = 2
_NUM_SUBCORES = 16
_NW = _NUM_CORES * _NUM_SUBCORES
_RPW = _B // _NW
_CHUNK = 128
_NCHUNK = _RPW // _CHUNK
_HIDDEN = 128
_OUT_DIM = 64


def _user_body(ut, iu, out, idx_v, ubuf, rows_u, sem):
    wid = lax.axis_index("s") * _NUM_CORES + lax.axis_index("c")
    base = wid * _RPW
    i16 = jnp.arange(16, dtype=jnp.int32)

    pltpu.sync_copy(iu.at[pl.ds(base, _RPW)], idx_v)

    for c in range(_RPW // 16):
        v16 = idx_v[pl.ds(c * 16, 16)]
        cps = []
        for l in range(16):
            v0 = pl.multiple_of((v16[l] >> 7) << 7, 128)
            cps.append(pltpu.async_copy(
                ut.at[pl.ds(0, 8), pl.ds(v0, 128)], ubuf.at[2 * l], sem))
            cps.append(pltpu.async_copy(
                ut.at[pl.ds(8, 8), pl.ds(v0, 128)], ubuf.at[2 * l + 1], sem))
        for cp in cps:
            cp.wait()
        for l in range(16):
            col = jnp.full((16,), 0, jnp.int32) + (v16[l] & 127)
            tsel = 2 * l + (i16 >= 8).astype(jnp.int32)
            val = plsc.load_gather(ubuf, [tsel, i16 & 7, col])
            plsc.store_scatter(rows_u, [i16 * _RPW + (c * 16 + l)], val)

    for e in range(16):
        pltpu.sync_copy(rows_u.at[pl.ds(e * _RPW, _RPW)],
                        out.at[e, pl.ds(base, _RPW)])


_user_gather = functools.partial(
    pl.kernel,
    mesh=plsc.VectorSubcoreMesh(core_axis_name="c", subcore_axis_name="s"),
    compiler_params=pltpu.CompilerParams(use_tc_tiling_on_sc=True,
                                         needs_layout_passes=False),
    out_type=jax.ShapeDtypeStruct((16, _B), jnp.float32),
    scratch_types=[
        pltpu.VMEM((_RPW,), jnp.int32),
        pltpu.VMEM((32, 8, 128), jnp.float32),
        pltpu.VMEM((16 * _RPW,), jnp.float32),
        pltpu.SemaphoreType.DMA,
    ],
)(_user_body)


_DIMS_B = (16, 8, 8, 8)
_OFFS_B = (0, 16 * _RPW, 24 * _RPW, 32 * _RPW)


def _rest_body(t0, t1, t2, t3, i0, i1, i2, i3, order_dep,
               o0, o1, o2, o3, cidx, gidx, idx_v, r1, r2, r3, rt, sem):
    del order_dep
    wid = lax.axis_index("s") * _NUM_CORES + lax.axis_index("c")
    row0 = wid * _NCHUNK
    base = wid * _RPW
    i16 = jnp.arange(16, dtype=jnp.int32)

    idx_refs = (i1, i2, i3)
    tables = (t1, t2, t3)
    rows = (r1, r2, r3)
    outs = (o0, o1, o2, o3)

    pltpu.sync_copy(i0.at[pl.ds(base, _RPW)], cidx)

    def build(r, _):
        vc = cidx[pl.ds(r * 16, 16)]
        for e in range(16):
            gidx[pl.ds(e * _RPW + r * 16, 16)] = vc + e * 100000
        return 0

    lax.fori_loop(0, _RPW // 16, build, 0)

    for f in range(3):
        pltpu.sync_copy(idx_refs[f].at[pl.ds(row0, _NCHUNK)], idx_v.at[f])

    cps = []
    for k in range(16 * _RPW // _CHUNK):
        cps.append(pltpu.async_copy(
            t0.at[gidx.at[pl.ds(k * _CHUNK, _CHUNK)]],
            rt.at[pl.ds(k * _CHUNK, _CHUNK)], sem))
    for f in range(3):
        for j in range(_NCHUNK):
            cps.append(pltpu.async_copy(
                tables[f].at[idx_v.at[f, j]],
                rows[f].at[pl.ds(j * _CHUNK, _CHUNK)], sem))
    for cp in cps:
        cp.wait()

    for f in range(3):
        d, off = _DIMS_B[f + 1], _OFFS_B[f + 1]

        def tr_body(r, _, f=f, d=d, off=off):
            ridx = r * 16 + i16
            for e in range(d):
                val = plsc.load_gather(
                    rows[f], [ridx, jnp.full((16,), e, jnp.int32)])
                rt[pl.ds(off + e * _RPW + r * 16, 16)] = val
            return 0

        lax.fori_loop(0, _RPW // 16, tr_body, 0)

    for f in range(4):
        d, off = _DIMS_B[f], _OFFS_B[f]
        for e in range(d):
            pltpu.sync_copy(rt.at[pl.ds(off + e * _RPW, _RPW)],
                            outs[f].at[e, pl.ds(base, _RPW)])


_rest_gather = functools.partial(
    pl.kernel,
    mesh=plsc.VectorSubcoreMesh(core_axis_name="c", subcore_axis_name="s"),
    compiler_params=pltpu.CompilerParams(use_tc_tiling_on_sc=False,
                                         needs_layout_passes=False),
    out_type=[jax.ShapeDtypeStruct((d, _B), jnp.float32)
              for d in _DIMS_B],
    scratch_types=[
        pltpu.VMEM((_RPW,), jnp.int32),
        pltpu.VMEM((16 * _RPW,), jnp.int32),
        pltpu.VMEM((3, _NCHUNK, _CHUNK), jnp.int32),
        pltpu.VMEM((_RPW, 8), jnp.float32),
        pltpu.VMEM((_RPW, 8), jnp.float32),
        pltpu.VMEM((_RPW, 8), jnp.float32),
        pltpu.VMEM((40 * _RPW,), jnp.float32),
        pltpu.SemaphoreType.DMA,
    ],
)(_rest_body)


def _mlp_body(xut, ec, ed, ea, eo, nt, w1u, w1c, w1d, w1a, w1o, w1n,
              b1, w2, b2, out):
    acc = lax.dot_general(w1u[...], xut[...], (((0,), (0,)), ((), ())),
                          preferred_element_type=jnp.float32)
    acc += lax.dot_general(w1c[...], ec[...], (((0,), (0,)), ((), ())),
                           preferred_element_type=jnp.float32)
    acc += lax.dot_general(w1d[...], ed[...], (((0,), (0,)), ((), ())),
                           preferred_element_type=jnp.float32)
    acc += lax.dot_general(w1a[...], ea[...], (((0,), (0,)), ((), ())),
                           preferred_element_type=jnp.float32)
    acc += lax.dot_general(w1o[...], eo[...], (((0,), (0,)), ((), ())),
                           preferred_element_type=jnp.float32)
    acc += lax.dot_general(w1n[...], nt[...], (((0,), (0,)), ((), ())),
                           preferred_element_type=jnp.float32)
    ht = jax.nn.relu(acc + b1[...])
    out[...] = jax.nn.relu(
        lax.dot_general(w2[...], ht, (((0,), (0,)), ((), ())),
                        preferred_element_type=jnp.float32) + b2[...])


def _mlp(xut, ec, ed, ea, eo, nt, w1u, w1c, w1d, w1a, w1o, w1n, b1, w2, b2):
    blk = 4096
    full = lambda s: pl.BlockSpec(s, lambda i: (0, 0))
    return pl.pallas_call(
        _mlp_body,
        grid=(_B // blk,),
        in_specs=[
            pl.BlockSpec((16, blk), lambda i: (0, i)),
            pl.BlockSpec((16, blk), lambda i: (0, i)),
            pl.BlockSpec((8, blk), lambda i: (0, i)),
            pl.BlockSpec((8, blk), lambda i: (0, i)),
            pl.BlockSpec((8, blk), lambda i: (0, i)),
            pl.BlockSpec((8, blk), lambda i: (0, i)),
            full((16, _HIDDEN)), full((16, _HIDDEN)), full((8, _HIDDEN)),
            full((8, _HIDDEN)), full((8, _HIDDEN)), full((8, _HIDDEN)),
            full((_HIDDEN, 1)), full((_HIDDEN, _OUT_DIM)), full((_OUT_DIM, 1)),
        ],
        out_specs=pl.BlockSpec((_OUT_DIM, blk), lambda i: (0, i)),
        out_shape=jax.ShapeDtypeStruct((_OUT_DIM, _B), jnp.float32),
    )(xut, ec, ed, ea, eo, nt, w1u, w1c, w1d, w1a, w1o, w1n, b1, w2, b2)


def kernel(user_id, city, device, age_bucket, occupation, user_num,
           E_user_id, E_city, E_device, E_age_bucket, E_occupation,
           W1, b1, W2, b2):
    xut = _user_gather(E_user_id.T, user_id)
    idx2d = [a.reshape(_NW * _NCHUNK, _CHUNK)
             for a in (device, age_bucket, occupation)]
    ec, ed, ea, eo = _rest_gather(
        E_city.T.reshape(-1), E_device, E_age_bucket, E_occupation,
        city, *idx2d, xut)
    ot = _mlp(xut, ec, ed, ea, eo, user_num.T,
              W1[0:16], W1[16:32], W1[32:40], W1[40:48], W1[48:56], W1[56:64],
              b1.reshape(_HIDDEN, 1), W2, b2.reshape(_OUT_DIM, 1))
    return ot.T
